# Initial kernel scaffold; baseline (speedup 1.0000x reference)
import jax
import jax.numpy as jnp
from jax.experimental import pallas as pl


def kernel(x, fine_edge_index, fine_angles, fine_transporters, coarse_idx, coarse_edge_index, coarse_angles, coarse_transporters, interp_matrix, e1, e2, W_embed, b_embed, Wc_self_0, Wc_nbr_0, b_c_0, Wc_self_1, Wc_nbr_1, b_c_1, Wc_self_2, Wc_nbr_2, b_c_2, Wc_self_3, Wc_nbr_3, b_c_3, Wc_self_4, Wc_nbr_4, b_c_4, Wc_self_5, Wc_nbr_5, b_c_5, W_csb, b_csb, W_cd1, b_cd1, W_cd2, b_cd2, W_cl1, b_cl1, W_cl2, b_cl2, W_up, b_up, Wr_self_0, Wr_nbr_0, b_r_0, Wr_self_1, Wr_nbr_1, b_r_1, W_psb, b_psb, W_cp1, b_cp1, W_cp2, b_cp2, W_wss):
    raise NotImplementedError("write your pallas kernel here")


# speedup vs baseline: 4.9884x; 4.9884x over previous
import functools

import jax
import jax.numpy as jnp
from jax import lax
from jax.experimental import pallas as pl
from jax.experimental.pallas import tpu as pltpu
from jax.experimental.pallas import tpu_sc as plsc

F32 = jnp.float32
I32 = jnp.int32

N_FINE = 10000
N_COARSE = 1000
E_FINE = 320000
E_COARSE = 32000
FD_RAW = 96
FD = 96

_SC_PARAMS = None


def _sc_compiler_params():
    return pltpu.CompilerParams(use_tc_tiling_on_sc=False)

NCORE = 2
NSUB = 16
NW = NCORE * NSUB

FK = 128
F_CHUNKS = 79
EF_PER_W = FK * F_CHUNKS
EF_PAD = EF_PER_W * NW

CK = 128
C_CHUNKS = 8
EC_PER_W = CK * C_CHUNKS
EC_PAD = EC_PER_W * NW

A_ROWS = 62500
A_ROWS_PAD = 62592
A_PER_SUB = A_ROWS_PAD // NSUB

PK = 64
P_CHUNKS = 5
NP_PER_W = PK * P_CHUNKS
NF_PAD = NP_PER_W * NW
NC_PAD = 1024

_HI = lax.Precision.HIGHEST


def _mm(a, b, precision=_HI):
    return jnp.dot(a, b, precision=precision, preferred_element_type=F32)


def _relu(v):
    return jnp.maximum(v, 0.0)


def _prep_body(ang_f, t0_f, t1_f, ang_c, t0_c, t1_c, src_c, dst_c, x,
               gf, gcos, gc, rid, lane, x16):
    a = ang_f[...]
    gf[...] = t0_f[...] * jnp.cos(a) + t1_f[...] * jnp.sin(a)
    gcos[...] = jnp.cos(a)
    ac = ang_c[...]
    gc[...] = t0_c[...] * jnp.cos(ac) + t1_c[...] * jnp.sin(ac)
    flat = dst_c[...] * N_COARSE + src_c[...]
    r = lax.shift_right_logical(flat, 4)
    rid[...] = r
    lane[...] = flat - (r * 16)
    xv = x[...]
    x16[...] = jnp.concatenate(
        [xv, jnp.ones((N_FINE, 1), F32), jnp.zeros((N_FINE, 11), F32)],
        axis=1)


_prep_call = pl.pallas_call(
    _prep_body,
    out_shape=(
        jax.ShapeDtypeStruct((E_FINE // 128, 128), F32),
        jax.ShapeDtypeStruct((E_FINE // 128, 128), F32),
        jax.ShapeDtypeStruct((E_COARSE // 128, 128), F32),
        jax.ShapeDtypeStruct((E_COARSE // 128, 128), I32),
        jax.ShapeDtypeStruct((E_COARSE // 128, 128), I32),
        jax.ShapeDtypeStruct((N_FINE, 16), F32),
    ),
)


@functools.cache
def _sc_mesh():
    return plsc.VectorSubcoreMesh(
        core_axis_name="c", subcore_axis_name="s",
        num_cores=NCORE, num_subcores=NSUB)


def _zero_fill(buf, n_rows, width):
    zer = jnp.zeros((16,), F32)

    def body(r, _):
        for q in range(width // 16):
            buf[r, pl.ds(q * 16, 16)] = zer
        return 0

    lax.fori_loop(0, n_rows, body, 0)


def _pool_body(x16_hbm, cidx_hbm, out_hbm, idx_v, rows_v, zero_v, acc_sh, sem):
    c = lax.axis_index("c")
    s = lax.axis_index("s")
    w = c * NSUB + s
    _zero_fill(zero_v, PK, 16)
    pltpu.sync_copy(zero_v, acc_sh.at[pl.ds(s * PK, PK)])
    plsc.subcore_barrier()
    pltpu.sync_copy(cidx_hbm.at[w], idx_v)

    def chunk(j, _):
        pltpu.async_copy(
            x16_hbm.at[pl.ds(w * NP_PER_W + j * PK, PK)], rows_v, sem).wait()
        pltpu.sync_copy(rows_v, acc_sh.at[idx_v.at[j]], add=True)
        return 0

    lax.fori_loop(0, P_CHUNKS, chunk, 0)
    plsc.subcore_barrier()
    pltpu.sync_copy(acc_sh.at[pl.ds(s * PK, PK)],
                    out_hbm.at[c, pl.ds(s * PK, PK)])


@functools.cache
def _pool_call():
    return functools.partial(
        pl.kernel,
        out_type=jax.ShapeDtypeStruct((NCORE, NC_PAD, 16), F32),
        mesh=_sc_mesh(),
        compiler_params=_sc_compiler_params(),
        scratch_types=[
            pltpu.VMEM((P_CHUNKS, PK), I32),
            pltpu.VMEM((PK, 16), F32),
            pltpu.VMEM((PK, 16), F32),
            pltpu.VMEM_SHARED((NC_PAD, 16), F32),
            pltpu.SemaphoreType.DMA,
        ],
    )(_pool_body)


def _adj_body(rid_hbm, lane_hbm, gate_hbm, out_hbm,
              rid_v, lane_v, gate_v, rows_v, zero_v, acc_sh, sem):
    c = lax.axis_index("c")
    s = lax.axis_index("s")
    w = c * NSUB + s
    _zero_fill(zero_v, CK, 16)
    base = s * A_PER_SUB

    def zrow(t, _):
        pltpu.sync_copy(zero_v, acc_sh.at[pl.ds(base + t * CK, CK)])
        return 0

    lax.fori_loop(0, 30, zrow, 0)
    pltpu.sync_copy(zero_v.at[pl.ds(0, 72)],
                    acc_sh.at[pl.ds(base + 30 * CK, 72)])
    plsc.subcore_barrier()

    pltpu.sync_copy(rid_hbm.at[w], rid_v)
    pltpu.sync_copy(lane_hbm.at[w], lane_v)
    pltpu.sync_copy(gate_hbm.at[w], gate_v)
    iota16 = lax.iota(I32, 16)
    zeros16 = jnp.zeros((16,), F32)

    def chunk(j, _):
        def group(t, _):
            l16 = lane_v[j, pl.ds(t * 16, 16)]
            g16 = gate_v[j, pl.ds(t * 16, 16)]
            for e in range(16):
                l_spl = jnp.full((16,), l16[e], I32)
                g_spl = jnp.full((16,), g16[e], F32)
                rows_v[t * 16 + e, :] = jnp.where(iota16 == l_spl, g_spl, zeros16)
            return 0

        lax.fori_loop(0, CK // 16, group, 0)
        pltpu.sync_copy(rows_v, acc_sh.at[rid_v.at[j]], add=True)
        return 0

    lax.fori_loop(0, C_CHUNKS, chunk, 0)
    plsc.subcore_barrier()

    def crow(t, _):
        pltpu.sync_copy(acc_sh.at[pl.ds(base + t * CK, CK)],
                        out_hbm.at[c, pl.ds(base + t * CK, CK)])
        return 0

    lax.fori_loop(0, 30, crow, 0)
    pltpu.sync_copy(acc_sh.at[pl.ds(base + 30 * CK, 72)],
                    out_hbm.at[c, pl.ds(base + 30 * CK, 72)])


@functools.cache
def _adj_call():
    return functools.partial(
        pl.kernel,
        out_type=jax.ShapeDtypeStruct((NCORE, A_ROWS_PAD, 16), F32),
        mesh=_sc_mesh(),
        compiler_params=_sc_compiler_params(),
        scratch_types=[
            pltpu.VMEM((C_CHUNKS, CK), I32),
            pltpu.VMEM((C_CHUNKS, CK), I32),
            pltpu.VMEM((C_CHUNKS, CK), F32),
            pltpu.VMEM((CK, 16), F32),
            pltpu.VMEM((CK, 16), F32),
            pltpu.VMEM_SHARED((A_ROWS_PAD, 16), F32),
            pltpu.SemaphoreType.DMA,
        ],
    )(_adj_body)


NF_ACC = 10112
_F_PER_SUB = NF_ACC // NSUB


def _fagg_body(h_hbm, src_hbm, dst_hbm, gate_hbm, out_hbm,
               src_v, dst_v, gate_v, rows_v, zero_v, acc_sh, sem):
    c = lax.axis_index("c")
    s = lax.axis_index("s")
    w = c * NSUB + s
    _zero_fill(zero_v, 128, FD)
    base = s * _F_PER_SUB

    def zrow(t, _):
        pltpu.sync_copy(zero_v, acc_sh.at[pl.ds(base + t * 128, 128)])
        return 0

    lax.fori_loop(0, 4, zrow, 0)
    pltpu.sync_copy(zero_v.at[pl.ds(0, 120)],
                    acc_sh.at[pl.ds(base + 4 * 128, 120)])
    plsc.subcore_barrier()

    pltpu.sync_copy(src_hbm.at[w], src_v)
    pltpu.sync_copy(dst_hbm.at[w], dst_v)
    pltpu.sync_copy(gate_hbm.at[w], gate_v)

    def chunk(j, _):
        pltpu.async_copy(h_hbm.at[src_v.at[j]], rows_v, sem).wait()

        def group(t, _):
            g16 = gate_v[j, pl.ds(t * 16, 16)]
            for e in range(16):
                g = jnp.full((16,), g16[e], F32)
                r = t * 16 + e
                for q in range(FD // 16):
                    sl = pl.ds(q * 16, 16)
                    rows_v[r, sl] = rows_v[r, sl] * g
            return 0

        lax.fori_loop(0, FK // 16, group, 0)
        pltpu.sync_copy(rows_v, acc_sh.at[dst_v.at[j]], add=True)
        return 0

    lax.fori_loop(0, F_CHUNKS, chunk, 0)
    plsc.subcore_barrier()

    def crow(t, _):
        pltpu.sync_copy(acc_sh.at[pl.ds(base + t * 128, 128)],
                        out_hbm.at[c, pl.ds(base + t * 128, 128)])
        return 0

    lax.fori_loop(0, 4, crow, 0)
    pltpu.sync_copy(acc_sh.at[pl.ds(base + 4 * 128, 120)],
                    out_hbm.at[c, pl.ds(base + 4 * 128, 120)])


@functools.cache
def _fagg_call():
    return functools.partial(
        pl.kernel,
        out_type=jax.ShapeDtypeStruct((NCORE, NF_ACC, FD), F32),
        mesh=_sc_mesh(),
        compiler_params=_sc_compiler_params(),
        scratch_types=[
            pltpu.VMEM((F_CHUNKS, FK), I32),
            pltpu.VMEM((F_CHUNKS, FK), I32),
            pltpu.VMEM((F_CHUNKS, FK), F32),
            pltpu.VMEM((FK, FD), F32),
            pltpu.VMEM((128, FD), F32),
            pltpu.VMEM_SHARED((NF_ACC, FD), F32),
            pltpu.SemaphoreType.DMA,
        ],
    )(_fagg_body)


def _coarse_body(pooled, a0, a1, W_embed, b_embed,
                 Ws0, Wn0, b0, Ws1, Wn1, b1, Ws2, Wn2, b2,
                 Ws3, Wn3, b3, Ws4, Wn4, b4, Ws5, Wn5, b5,
                 W_csb, b_csb, W_cd1, b_cd1, W_cd2, b_cd2,
                 W_cl1, b_cl1, W_cl2, b_cl2, W_up192,
                 w1_out, cd_out, cl_out):
    ps = pooled[0] + pooled[1]
    sums4 = ps[0:N_COARSE, 0:4]
    cnt_raw = ps[0:N_COARSE, 4:5]
    cnt = jnp.maximum(cnt_raw, 1.0)
    h = _mm(sums4, W_embed[...]) / cnt + jnp.minimum(cnt_raw, 1.0) * b_embed[...]
    A = a0[...] + a1[...]
    blocks = ((Ws0, Wn0, b0), (Ws1, Wn1, b1), (Ws2, Wn2, b2),
              (Ws3, Wn3, b3), (Ws4, Wn4, b4), (Ws5, Wn5, b5))
    for Ws, Wn, b in blocks:
        din, dout = Ws.shape
        if dout <= din:
            agg = _mm(A, _mm(h, Wn[...]))
        else:
            agg = _mm(_mm(A, h), Wn[...])
        h = _relu(_mm(h, Ws[...]) + agg + b[...])
    h_cs = _mm(h, W_csb[...]) + b_csb[...]
    pv = jnp.mean(h_cs, axis=0, keepdims=True)
    cd_out[...] = _mm(_relu(_mm(pv, W_cd1[...]) + b_cd1[...]), W_cd2[...]) + b_cd2[...]
    cl_out[...] = _mm(_relu(_mm(pv, W_cl1[...]) + b_cl1[...]), W_cl2[...]) + b_cl2[...]
    w1_out[...] = _mm(h, W_up192[...])


_coarse_call = pl.pallas_call(
    _coarse_body,
    out_shape=(
        jax.ShapeDtypeStruct((N_COARSE, FD_RAW), F32),
        jax.ShapeDtypeStruct((1, 1), F32),
        jax.ShapeDtypeStruct((1, 1), F32),
    ),
)


_IROWS = 1000


def _interp_body(im, x16, w1, w2, b_up, out):
    acc = _mm(im[...], w1[...], precision=lax.Precision.DEFAULT)
    out[...] = _relu(acc + _mm(x16[...], w2[...]) + b_up[...])


_interp_call = pl.pallas_call(
    _interp_body,
    grid=(N_FINE // _IROWS,),
    in_specs=[
        pl.BlockSpec((_IROWS, N_COARSE), lambda i: (i, 0)),
        pl.BlockSpec((_IROWS, 16), lambda i: (i, 0)),
        pl.BlockSpec((N_COARSE, FD), lambda i: (0, 0)),
        pl.BlockSpec((16, FD), lambda i: (0, 0)),
        pl.BlockSpec((1, FD), lambda i: (0, 0)),
    ],
    out_specs=pl.BlockSpec((_IROWS, FD), lambda i: (i, 0)),
    out_shape=jax.ShapeDtypeStruct((N_FINE, FD), F32),
)


def _fupd_body(h, p0, p1, Ws, Wn, b, out):
    out[...] = _relu(_mm(h[...], Ws[...]) + _mm(p0[...] + p1[...], Wn[...]) + b[...])


_FR = 2000

_fupd_call = pl.pallas_call(
    _fupd_body,
    grid=(N_FINE // _FR,),
    in_specs=[
        pl.BlockSpec((_FR, FD), lambda i: (i, 0)),
        pl.BlockSpec((_FR, FD), lambda i: (i, 0)),
        pl.BlockSpec((_FR, FD), lambda i: (i, 0)),
        pl.BlockSpec((FD, FD), lambda i: (0, 0)),
        pl.BlockSpec((FD, FD), lambda i: (0, 0)),
        pl.BlockSpec((1, FD), lambda i: (0, 0)),
    ],
    out_specs=pl.BlockSpec((_FR, FD), lambda i: (i, 0)),
    out_shape=jax.ShapeDtypeStruct((N_FINE, FD), F32),
)


def _heads_body(h, p0, p1, W_psb, b_psb, W_cp1, b_cp1, W_cp2, b_cp2,
                W_wss, e1, e2, cp_out, wss_out):
    h_cp = _mm(h[...], W_psb[...]) + b_psb[...]
    cp_out[...] = _mm(_relu(_mm(h_cp, W_cp1[...]) + b_cp1[...]), W_cp2[...]) + b_cp2[...]
    coeffs = _mm(p0[...] + p1[...], W_wss[...])
    wss_out[...] = coeffs[:, 0:1] * e1[...] + coeffs[:, 1:2] * e2[...]


_heads_call = pl.pallas_call(
    _heads_body,
    grid=(N_FINE // _FR,),
    in_specs=[
        pl.BlockSpec((_FR, FD), lambda i: (i, 0)),
        pl.BlockSpec((_FR, FD), lambda i: (i, 0)),
        pl.BlockSpec((_FR, FD), lambda i: (i, 0)),
        pl.BlockSpec((FD, 64), lambda i: (0, 0)),
        pl.BlockSpec((1, 64), lambda i: (0, 0)),
        pl.BlockSpec((64, 128), lambda i: (0, 0)),
        pl.BlockSpec((1, 128), lambda i: (0, 0)),
        pl.BlockSpec((128, 1), lambda i: (0, 0)),
        pl.BlockSpec((1, 1), lambda i: (0, 0)),
        pl.BlockSpec((FD, 2), lambda i: (0, 0)),
        pl.BlockSpec((_FR, 3), lambda i: (i, 0)),
        pl.BlockSpec((_FR, 3), lambda i: (i, 0)),
    ],
    out_specs=(
        pl.BlockSpec((_FR, 1), lambda i: (i, 0)),
        pl.BlockSpec((_FR, 3), lambda i: (i, 0)),
    ),
    out_shape=(
        jax.ShapeDtypeStruct((N_FINE, 1), F32),
        jax.ShapeDtypeStruct((N_FINE, 3), F32),
    ),
)


def _pad_worker(arr, per_w, chunks, k, dtype):
    n = arr.shape[0]
    arr = jnp.pad(arr, (0, per_w * NW - n))
    return arr.reshape(NW, chunks, k).astype(dtype)


def kernel(x, fine_edge_index, fine_angles, fine_transporters, coarse_idx,
           coarse_edge_index, coarse_angles, coarse_transporters, interp_matrix,
           e1, e2, W_embed, b_embed,
           Wc_self_0, Wc_nbr_0, b_c_0, Wc_self_1, Wc_nbr_1, b_c_1,
           Wc_self_2, Wc_nbr_2, b_c_2, Wc_self_3, Wc_nbr_3, b_c_3,
           Wc_self_4, Wc_nbr_4, b_c_4, Wc_self_5, Wc_nbr_5, b_c_5,
           W_csb, b_csb, W_cd1, b_cd1, W_cd2, b_cd2,
           W_cl1, b_cl1, W_cl2, b_cl2, W_up, b_up,
           Wr_self_0, Wr_nbr_0, b_r_0, Wr_self_1, Wr_nbr_1, b_r_1,
           W_psb, b_psb, W_cp1, b_cp1, W_cp2, b_cp2, W_wss):
    row = lambda v: v.reshape(1, -1)

    ang_f = fine_angles.reshape(E_FINE // 128, 128)
    t0_f = fine_transporters[:, 0].reshape(E_FINE // 128, 128)
    t1_f = fine_transporters[:, 1].reshape(E_FINE // 128, 128)
    ang_c = coarse_angles.reshape(E_COARSE // 128, 128)
    t0_c = coarse_transporters[:, 0].reshape(E_COARSE // 128, 128)
    t1_c = coarse_transporters[:, 1].reshape(E_COARSE // 128, 128)
    src_c = coarse_edge_index[0].reshape(E_COARSE // 128, 128)
    dst_c = coarse_edge_index[1].reshape(E_COARSE // 128, 128)
    gf, gcos, gc, rid, lane, x16 = _prep_call(
        ang_f, t0_f, t1_f, ang_c, t0_c, t1_c, src_c, dst_c, x)

    x16_pad = jnp.pad(x16, ((0, NF_PAD - N_FINE), (0, 0)))
    cidx = _pad_worker(coarse_idx, NP_PER_W, P_CHUNKS, PK, I32)
    pooled = _pool_call()(x16_pad, cidx)

    rid_w = _pad_worker(rid.reshape(-1), EC_PER_W, C_CHUNKS, CK, I32)
    lane_w = _pad_worker(lane.reshape(-1), EC_PER_W, C_CHUNKS, CK, I32)
    gc_w = _pad_worker(gc.reshape(-1), EC_PER_W, C_CHUNKS, CK, F32)
    a_parts = _adj_call()(rid_w, lane_w, gc_w)
    a_flat = a_parts.reshape(NCORE, A_ROWS_PAD * 16)[:, :N_COARSE * N_COARSE]
    a0 = a_flat[0].reshape(N_COARSE, N_COARSE)
    a1 = a_flat[1].reshape(N_COARSE, N_COARSE)

    w1, cd, cl = _coarse_call(
        pooled, a0, a1, W_embed, row(b_embed),
        Wc_self_0, Wc_nbr_0, row(b_c_0), Wc_self_1, Wc_nbr_1, row(b_c_1),
        Wc_self_2, Wc_nbr_2, row(b_c_2), Wc_self_3, Wc_nbr_3, row(b_c_3),
        Wc_self_4, Wc_nbr_4, row(b_c_4), Wc_self_5, Wc_nbr_5, row(b_c_5),
        W_csb, row(b_csb), W_cd1, row(b_cd1), W_cd2, row(b_cd2),
        W_cl1, row(b_cl1), W_cl2, row(b_cl2), W_up[0:192, :])

    pad_c = FD - FD_RAW
    w1p = jnp.pad(w1, ((0, 0), (0, pad_c)))
    w2 = jnp.pad(W_up[192:196, :], ((0, 12), (0, pad_c)))
    h_f = _interp_call(interp_matrix, x16, w1p, w2, row(jnp.pad(b_up, (0, pad_c))))

    src_w = _pad_worker(fine_edge_index[0], EF_PER_W, F_CHUNKS, FK, I32)
    dst_w = _pad_worker(fine_edge_index[1], EF_PER_W, F_CHUNKS, FK, I32)
    gf_w = _pad_worker(gf.reshape(-1), EF_PER_W, F_CHUNKS, FK, F32)
    for Ws, Wn, b in ((Wr_self_0, Wr_nbr_0, b_r_0),
                      (Wr_self_1, Wr_nbr_1, b_r_1)):
        parts = _fagg_call()(h_f, src_w, dst_w, gf_w)
        h_f = _fupd_call(h_f, parts[0, :N_FINE], parts[1, :N_FINE],
                         jnp.pad(Ws, ((0, pad_c), (0, pad_c))),
                         jnp.pad(Wn, ((0, 0), (0, pad_c))),
                         row(jnp.pad(b, (0, pad_c))))

    gcos_w = _pad_worker(gcos.reshape(-1), EF_PER_W, F_CHUNKS, FK, F32)
    parts = _fagg_call()(h_f, src_w, dst_w, gcos_w)

    cp, wss = _heads_call(
        h_f, parts[0, :N_FINE], parts[1, :N_FINE],
        jnp.pad(W_psb, ((0, pad_c), (0, 0))), row(b_psb), W_cp1, row(b_cp1),
        W_cp2, row(b_cp2), W_wss, e1, e2)

    return {'cp': cp.reshape(N_FINE),
            'wss': wss,
            'cd': cd.reshape(1),
            'cl': cl.reshape(1)}


# speedup vs baseline: 6.6235x; 1.3278x over previous
import functools

import jax
import jax.numpy as jnp
from jax import lax
from jax.experimental import pallas as pl
from jax.experimental.pallas import tpu as pltpu
from jax.experimental.pallas import tpu_sc as plsc

F32 = jnp.float32
I32 = jnp.int32

N_FINE = 10000
N_COARSE = 1000
E_FINE = 320000
E_COARSE = 32000
FD_RAW = 96
FD = 96

_SC_PARAMS = None


def _sc_compiler_params():
    return pltpu.CompilerParams(use_tc_tiling_on_sc=False)

NCORE = 2
NSUB = 16
NW = NCORE * NSUB

FK = 128
F_CHUNKS = 79
EF_PER_W = FK * F_CHUNKS
EF_PAD = EF_PER_W * NW

CK = 128
C_CHUNKS = 8
EC_PER_W = CK * C_CHUNKS
EC_PAD = EC_PER_W * NW

A_ROWS = 62500
A_ROWS_PAD = 62592
A_PER_SUB = A_ROWS_PAD // NSUB

PK = 64
P_CHUNKS = 5
NP_PER_W = PK * P_CHUNKS
NF_PAD = NP_PER_W * NW
NC_PAD = 1024

def _mm(a, b, precision=None):
    return jnp.dot(a, b, precision=precision, preferred_element_type=F32)


def _relu(v):
    return jnp.maximum(v, 0.0)


def _prep_body(ang_f, t0_f, t1_f, ang_c, t0_c, t1_c, src_c, dst_c, x,
               gf, gcos, gc, rid, lane, x16):
    a = ang_f[...]
    gf[...] = t0_f[...] * jnp.cos(a) + t1_f[...] * jnp.sin(a)
    gcos[...] = jnp.cos(a)
    ac = ang_c[...]
    gc[...] = t0_c[...] * jnp.cos(ac) + t1_c[...] * jnp.sin(ac)
    flat = dst_c[...] * N_COARSE + src_c[...]
    r = lax.shift_right_logical(flat, 4)
    rid[...] = r
    lane[...] = flat - (r * 16)
    xv = x[...]
    x16[...] = jnp.concatenate(
        [xv, jnp.ones((N_FINE, 1), F32), jnp.zeros((N_FINE, 11), F32)],
        axis=1)


_prep_call = pl.pallas_call(
    _prep_body,
    out_shape=(
        jax.ShapeDtypeStruct((E_FINE // 128, 128), F32),
        jax.ShapeDtypeStruct((E_FINE // 128, 128), F32),
        jax.ShapeDtypeStruct((E_COARSE // 128, 128), F32),
        jax.ShapeDtypeStruct((E_COARSE // 128, 128), I32),
        jax.ShapeDtypeStruct((E_COARSE // 128, 128), I32),
        jax.ShapeDtypeStruct((N_FINE, 16), F32),
    ),
)


@functools.cache
def _sc_mesh():
    return plsc.VectorSubcoreMesh(
        core_axis_name="c", subcore_axis_name="s",
        num_cores=NCORE, num_subcores=NSUB)


def _zero_fill(buf, n_rows, width):
    zer = jnp.zeros((16,), F32)

    def body(r, _):
        for q in range(width // 16):
            buf[r, pl.ds(q * 16, 16)] = zer
        return 0

    lax.fori_loop(0, n_rows, body, 0)


def _pool_body(x16_hbm, cidx_hbm, out_hbm, idx_v, rows_v, zero_v, acc_sh, sem):
    c = lax.axis_index("c")
    s = lax.axis_index("s")
    w = c * NSUB + s
    _zero_fill(zero_v, PK, 16)
    pltpu.sync_copy(zero_v, acc_sh.at[pl.ds(s * PK, PK)])
    plsc.subcore_barrier()
    pltpu.sync_copy(cidx_hbm.at[w], idx_v)

    def chunk(j, _):
        pltpu.async_copy(
            x16_hbm.at[pl.ds(w * NP_PER_W + j * PK, PK)], rows_v, sem).wait()
        pltpu.sync_copy(rows_v, acc_sh.at[idx_v.at[j]], add=True)
        return 0

    lax.fori_loop(0, P_CHUNKS, chunk, 0)
    plsc.subcore_barrier()
    pltpu.sync_copy(acc_sh.at[pl.ds(s * PK, PK)],
                    out_hbm.at[c, pl.ds(s * PK, PK)])


@functools.cache
def _pool_call():
    return functools.partial(
        pl.kernel,
        out_type=jax.ShapeDtypeStruct((NCORE, NC_PAD, 16), F32),
        mesh=_sc_mesh(),
        compiler_params=_sc_compiler_params(),
        scratch_types=[
            pltpu.VMEM((P_CHUNKS, PK), I32),
            pltpu.VMEM((PK, 16), F32),
            pltpu.VMEM((PK, 16), F32),
            pltpu.VMEM_SHARED((NC_PAD, 16), F32),
            pltpu.SemaphoreType.DMA,
        ],
    )(_pool_body)


def _adj_body(rid_hbm, lane_hbm, gate_hbm, out_hbm,
              rid_v, lane_v, gate_v, rows_v, zero_v, acc_sh, sem):
    c = lax.axis_index("c")
    s = lax.axis_index("s")
    w = c * NSUB + s
    _zero_fill(zero_v, CK, 16)
    base = s * A_PER_SUB

    def zrow(t, _):
        pltpu.sync_copy(zero_v, acc_sh.at[pl.ds(base + t * CK, CK)])
        return 0

    lax.fori_loop(0, 30, zrow, 0)
    pltpu.sync_copy(zero_v.at[pl.ds(0, 72)],
                    acc_sh.at[pl.ds(base + 30 * CK, 72)])
    plsc.subcore_barrier()

    pltpu.sync_copy(rid_hbm.at[w], rid_v)
    pltpu.sync_copy(lane_hbm.at[w], lane_v)
    pltpu.sync_copy(gate_hbm.at[w], gate_v)
    iota16 = lax.iota(I32, 16)
    zeros16 = jnp.zeros((16,), F32)

    def chunk(j, _):
        def group(t, _):
            l16 = lane_v[j, pl.ds(t * 16, 16)]
            g16 = gate_v[j, pl.ds(t * 16, 16)]
            for e in range(16):
                l_spl = jnp.full((16,), l16[e], I32)
                g_spl = jnp.full((16,), g16[e], F32)
                rows_v[t * 16 + e, :] = jnp.where(iota16 == l_spl, g_spl, zeros16)
            return 0

        lax.fori_loop(0, CK // 16, group, 0)
        pltpu.sync_copy(rows_v, acc_sh.at[rid_v.at[j]], add=True)
        return 0

    lax.fori_loop(0, C_CHUNKS, chunk, 0)
    plsc.subcore_barrier()

    def crow(t, _):
        pltpu.sync_copy(acc_sh.at[pl.ds(base + t * CK, CK)],
                        out_hbm.at[c, pl.ds(base + t * CK, CK)])
        return 0

    lax.fori_loop(0, 30, crow, 0)
    pltpu.sync_copy(acc_sh.at[pl.ds(base + 30 * CK, 72)],
                    out_hbm.at[c, pl.ds(base + 30 * CK, 72)])


@functools.cache
def _adj_call():
    return functools.partial(
        pl.kernel,
        out_type=jax.ShapeDtypeStruct((NCORE, A_ROWS_PAD, 16), F32),
        mesh=_sc_mesh(),
        compiler_params=_sc_compiler_params(),
        scratch_types=[
            pltpu.VMEM((C_CHUNKS, CK), I32),
            pltpu.VMEM((C_CHUNKS, CK), I32),
            pltpu.VMEM((C_CHUNKS, CK), F32),
            pltpu.VMEM((CK, 16), F32),
            pltpu.VMEM((CK, 16), F32),
            pltpu.VMEM_SHARED((A_ROWS_PAD, 16), F32),
            pltpu.SemaphoreType.DMA,
        ],
    )(_adj_body)


NF_ACC = 10112
_F_PER_SUB = NF_ACC // NSUB


def _fagg_body(h_hbm, src_hbm, dst_hbm, gate_hbm, out_hbm,
               src_v, dst_v, gate_v, rows_v, rows_b, zero_v, acc_sh, sem, sem_b):
    c = lax.axis_index("c")
    s = lax.axis_index("s")
    w = c * NSUB + s
    _zero_fill(zero_v, 128, FD)
    base = s * _F_PER_SUB

    def zrow(t, _):
        pltpu.sync_copy(zero_v, acc_sh.at[pl.ds(base + t * 128, 128)])
        return 0

    lax.fori_loop(0, 4, zrow, 0)
    pltpu.sync_copy(zero_v.at[pl.ds(0, 120)],
                    acc_sh.at[pl.ds(base + 4 * 128, 120)])
    plsc.subcore_barrier()

    pltpu.sync_copy(src_hbm.at[w], src_v)
    pltpu.sync_copy(dst_hbm.at[w], dst_v)
    pltpu.sync_copy(gate_hbm.at[w], gate_v)

    def scale_and_scatter(j, rows, sem_g):
        pltpu.make_async_copy(h_hbm.at[src_v.at[j]], rows, sem_g).wait()

        def group(t, _):
            g16 = gate_v[j, pl.ds(t * 16, 16)]
            for e in range(16):
                g = jnp.full((16,), g16[e], F32)
                r = t * 16 + e
                for q in range(FD // 16):
                    sl = pl.ds(q * 16, 16)
                    rows[r, sl] = rows[r, sl] * g
            return 0

        lax.fori_loop(0, FK // 16, group, 0)
        pltpu.sync_copy(rows, acc_sh.at[dst_v.at[j]], add=True)

    pltpu.async_copy(h_hbm.at[src_v.at[0]], rows_v, sem)

    def pair(k, _):
        a = 2 * k
        pltpu.async_copy(h_hbm.at[src_v.at[a + 1]], rows_b, sem_b)
        scale_and_scatter(a, rows_v, sem)
        pltpu.async_copy(h_hbm.at[src_v.at[a + 2]], rows_v, sem)
        scale_and_scatter(a + 1, rows_b, sem_b)
        return 0

    lax.fori_loop(0, (F_CHUNKS - 1) // 2, pair, 0)
    scale_and_scatter(F_CHUNKS - 1, rows_v, sem)
    plsc.subcore_barrier()

    def crow(t, _):
        pltpu.sync_copy(acc_sh.at[pl.ds(base + t * 128, 128)],
                        out_hbm.at[c, pl.ds(base + t * 128, 128)])
        return 0

    lax.fori_loop(0, 4, crow, 0)
    pltpu.sync_copy(acc_sh.at[pl.ds(base + 4 * 128, 120)],
                    out_hbm.at[c, pl.ds(base + 4 * 128, 120)])


@functools.cache
def _fagg_call():
    return functools.partial(
        pl.kernel,
        out_type=jax.ShapeDtypeStruct((NCORE, NF_ACC, FD), F32),
        mesh=_sc_mesh(),
        compiler_params=_sc_compiler_params(),
        scratch_types=[
            pltpu.VMEM((F_CHUNKS, FK), I32),
            pltpu.VMEM((F_CHUNKS, FK), I32),
            pltpu.VMEM((F_CHUNKS, FK), F32),
            pltpu.VMEM((FK, FD), F32),
            pltpu.VMEM((FK, FD), F32),
            pltpu.VMEM((128, FD), F32),
            pltpu.VMEM_SHARED((NF_ACC, FD), F32),
            pltpu.SemaphoreType.DMA,
            pltpu.SemaphoreType.DMA,
        ],
    )(_fagg_body)


def _coarse_body(pooled, a0, a1, W_embed, b_embed,
                 Ws0, Wn0, b0, Ws1, Wn1, b1, Ws2, Wn2, b2,
                 Ws3, Wn3, b3, Ws4, Wn4, b4, Ws5, Wn5, b5,
                 W_csb, b_csb, W_cd1, b_cd1, W_cd2, b_cd2,
                 W_cl1, b_cl1, W_cl2, b_cl2, W_up192,
                 w1_out, cd_out, cl_out):
    ps = pooled[0] + pooled[1]
    sums4 = ps[0:N_COARSE, 0:4]
    cnt_raw = ps[0:N_COARSE, 4:5]
    cnt = jnp.maximum(cnt_raw, 1.0)
    h = _mm(sums4, W_embed[...]) / cnt + jnp.minimum(cnt_raw, 1.0) * b_embed[...]
    A = a0[...] + a1[...]
    blocks = ((Ws0, Wn0, b0), (Ws1, Wn1, b1), (Ws2, Wn2, b2),
              (Ws3, Wn3, b3), (Ws4, Wn4, b4), (Ws5, Wn5, b5))
    for Ws, Wn, b in blocks:
        din, dout = Ws.shape
        if dout <= din:
            agg = _mm(A, _mm(h, Wn[...]))
        else:
            agg = _mm(_mm(A, h), Wn[...])
        h = _relu(_mm(h, Ws[...]) + agg + b[...])
    h_cs = _mm(h, W_csb[...]) + b_csb[...]
    pv = jnp.mean(h_cs, axis=0, keepdims=True)
    cd_out[...] = _mm(_relu(_mm(pv, W_cd1[...]) + b_cd1[...]), W_cd2[...]) + b_cd2[...]
    cl_out[...] = _mm(_relu(_mm(pv, W_cl1[...]) + b_cl1[...]), W_cl2[...]) + b_cl2[...]
    w1_out[...] = _mm(h, W_up192[...])


_coarse_call = pl.pallas_call(
    _coarse_body,
    out_shape=(
        jax.ShapeDtypeStruct((N_COARSE, FD_RAW), F32),
        jax.ShapeDtypeStruct((1, 1), F32),
        jax.ShapeDtypeStruct((1, 1), F32),
    ),
)


_IROWS = 1000


def _interp_body(im, x16, w1, w2, b_up, out):
    acc = _mm(im[...], w1[...], precision=lax.Precision.DEFAULT)
    out[...] = _relu(acc + _mm(x16[...], w2[...]) + b_up[...])


_interp_call = pl.pallas_call(
    _interp_body,
    grid=(N_FINE // _IROWS,),
    in_specs=[
        pl.BlockSpec((_IROWS, N_COARSE), lambda i: (i, 0)),
        pl.BlockSpec((_IROWS, 16), lambda i: (i, 0)),
        pl.BlockSpec((N_COARSE, FD), lambda i: (0, 0)),
        pl.BlockSpec((16, FD), lambda i: (0, 0)),
        pl.BlockSpec((1, FD), lambda i: (0, 0)),
    ],
    out_specs=pl.BlockSpec((_IROWS, FD), lambda i: (i, 0)),
    out_shape=jax.ShapeDtypeStruct((N_FINE, FD), F32),
)


def _fupd_body(h, p0, p1, Ws, Wn, b, out):
    out[...] = _relu(_mm(h[...], Ws[...]) + _mm(p0[...] + p1[...], Wn[...]) + b[...])


_FR = 2000

_fupd_call = pl.pallas_call(
    _fupd_body,
    grid=(N_FINE // _FR,),
    in_specs=[
        pl.BlockSpec((_FR, FD), lambda i: (i, 0)),
        pl.BlockSpec((_FR, FD), lambda i: (i, 0)),
        pl.BlockSpec((_FR, FD), lambda i: (i, 0)),
        pl.BlockSpec((FD, FD), lambda i: (0, 0)),
        pl.BlockSpec((FD, FD), lambda i: (0, 0)),
        pl.BlockSpec((1, FD), lambda i: (0, 0)),
    ],
    out_specs=pl.BlockSpec((_FR, FD), lambda i: (i, 0)),
    out_shape=jax.ShapeDtypeStruct((N_FINE, FD), F32),
)


def _heads_body(h, p0, p1, W_psb, b_psb, W_cp1, b_cp1, W_cp2, b_cp2,
                W_wss, e1, e2, cp_out, wss_out):
    h_cp = _mm(h[...], W_psb[...]) + b_psb[...]
    cp_out[...] = _mm(_relu(_mm(h_cp, W_cp1[...]) + b_cp1[...]), W_cp2[...]) + b_cp2[...]
    coeffs = _mm(p0[...] + p1[...], W_wss[...])
    wss_out[...] = coeffs[:, 0:1] * e1[...] + coeffs[:, 1:2] * e2[...]


_heads_call = pl.pallas_call(
    _heads_body,
    grid=(N_FINE // _FR,),
    in_specs=[
        pl.BlockSpec((_FR, FD), lambda i: (i, 0)),
        pl.BlockSpec((_FR, FD), lambda i: (i, 0)),
        pl.BlockSpec((_FR, FD), lambda i: (i, 0)),
        pl.BlockSpec((FD, 64), lambda i: (0, 0)),
        pl.BlockSpec((1, 64), lambda i: (0, 0)),
        pl.BlockSpec((64, 128), lambda i: (0, 0)),
        pl.BlockSpec((1, 128), lambda i: (0, 0)),
        pl.BlockSpec((128, 1), lambda i: (0, 0)),
        pl.BlockSpec((1, 1), lambda i: (0, 0)),
        pl.BlockSpec((FD, 2), lambda i: (0, 0)),
        pl.BlockSpec((_FR, 3), lambda i: (i, 0)),
        pl.BlockSpec((_FR, 3), lambda i: (i, 0)),
    ],
    out_specs=(
        pl.BlockSpec((_FR, 1), lambda i: (i, 0)),
        pl.BlockSpec((_FR, 3), lambda i: (i, 0)),
    ),
    out_shape=(
        jax.ShapeDtypeStruct((N_FINE, 1), F32),
        jax.ShapeDtypeStruct((N_FINE, 3), F32),
    ),
)


def _pad_worker(arr, per_w, chunks, k, dtype):
    n = arr.shape[0]
    arr = jnp.pad(arr, (0, per_w * NW - n))
    return arr.reshape(NW, chunks, k).astype(dtype)


def kernel(x, fine_edge_index, fine_angles, fine_transporters, coarse_idx,
           coarse_edge_index, coarse_angles, coarse_transporters, interp_matrix,
           e1, e2, W_embed, b_embed,
           Wc_self_0, Wc_nbr_0, b_c_0, Wc_self_1, Wc_nbr_1, b_c_1,
           Wc_self_2, Wc_nbr_2, b_c_2, Wc_self_3, Wc_nbr_3, b_c_3,
           Wc_self_4, Wc_nbr_4, b_c_4, Wc_self_5, Wc_nbr_5, b_c_5,
           W_csb, b_csb, W_cd1, b_cd1, W_cd2, b_cd2,
           W_cl1, b_cl1, W_cl2, b_cl2, W_up, b_up,
           Wr_self_0, Wr_nbr_0, b_r_0, Wr_self_1, Wr_nbr_1, b_r_1,
           W_psb, b_psb, W_cp1, b_cp1, W_cp2, b_cp2, W_wss):
    row = lambda v: v.reshape(1, -1)

    ang_f = fine_angles.reshape(E_FINE // 128, 128)
    t0_f = fine_transporters[:, 0].reshape(E_FINE // 128, 128)
    t1_f = fine_transporters[:, 1].reshape(E_FINE // 128, 128)
    ang_c = coarse_angles.reshape(E_COARSE // 128, 128)
    t0_c = coarse_transporters[:, 0].reshape(E_COARSE // 128, 128)
    t1_c = coarse_transporters[:, 1].reshape(E_COARSE // 128, 128)
    src_c = coarse_edge_index[0].reshape(E_COARSE // 128, 128)
    dst_c = coarse_edge_index[1].reshape(E_COARSE // 128, 128)
    gf, gcos, gc, rid, lane, x16 = _prep_call(
        ang_f, t0_f, t1_f, ang_c, t0_c, t1_c, src_c, dst_c, x)

    x16_pad = jnp.pad(x16, ((0, NF_PAD - N_FINE), (0, 0)))
    cidx = _pad_worker(coarse_idx, NP_PER_W, P_CHUNKS, PK, I32)
    pooled = _pool_call()(x16_pad, cidx)

    rid_w = _pad_worker(rid.reshape(-1), EC_PER_W, C_CHUNKS, CK, I32)
    lane_w = _pad_worker(lane.reshape(-1), EC_PER_W, C_CHUNKS, CK, I32)
    gc_w = _pad_worker(gc.reshape(-1), EC_PER_W, C_CHUNKS, CK, F32)
    a_parts = _adj_call()(rid_w, lane_w, gc_w)
    a_flat = a_parts.reshape(NCORE, A_ROWS_PAD * 16)[:, :N_COARSE * N_COARSE]
    a0 = a_flat[0].reshape(N_COARSE, N_COARSE)
    a1 = a_flat[1].reshape(N_COARSE, N_COARSE)

    w1, cd, cl = _coarse_call(
        pooled, a0, a1, W_embed, row(b_embed),
        Wc_self_0, Wc_nbr_0, row(b_c_0), Wc_self_1, Wc_nbr_1, row(b_c_1),
        Wc_self_2, Wc_nbr_2, row(b_c_2), Wc_self_3, Wc_nbr_3, row(b_c_3),
        Wc_self_4, Wc_nbr_4, row(b_c_4), Wc_self_5, Wc_nbr_5, row(b_c_5),
        W_csb, row(b_csb), W_cd1, row(b_cd1), W_cd2, row(b_cd2),
        W_cl1, row(b_cl1), W_cl2, row(b_cl2), W_up[0:192, :])

    pad_c = FD - FD_RAW
    w1p = jnp.pad(w1, ((0, 0), (0, pad_c)))
    w2 = jnp.pad(W_up[192:196, :], ((0, 12), (0, pad_c)))
    h_f = _interp_call(interp_matrix, x16, w1p, w2, row(jnp.pad(b_up, (0, pad_c))))

    src_w = _pad_worker(fine_edge_index[0], EF_PER_W, F_CHUNKS, FK, I32)
    dst_w = _pad_worker(fine_edge_index[1], EF_PER_W, F_CHUNKS, FK, I32)
    gf_w = _pad_worker(gf.reshape(-1), EF_PER_W, F_CHUNKS, FK, F32)
    for Ws, Wn, b in ((Wr_self_0, Wr_nbr_0, b_r_0),
                      (Wr_self_1, Wr_nbr_1, b_r_1)):
        parts = _fagg_call()(h_f, src_w, dst_w, gf_w)
        h_f = _fupd_call(h_f, parts[0, :N_FINE], parts[1, :N_FINE],
                         jnp.pad(Ws, ((0, pad_c), (0, pad_c))),
                         jnp.pad(Wn, ((0, 0), (0, pad_c))),
                         row(jnp.pad(b, (0, pad_c))))

    gcos_w = _pad_worker(gcos.reshape(-1), EF_PER_W, F_CHUNKS, FK, F32)
    parts = _fagg_call()(h_f, src_w, dst_w, gcos_w)

    cp, wss = _heads_call(
        h_f, parts[0, :N_FINE], parts[1, :N_FINE],
        jnp.pad(W_psb, ((0, pad_c), (0, 0))), row(b_psb), W_cp1, row(b_cp1),
        W_cp2, row(b_cp2), W_wss, e1, e2)

    return {'cp': cp.reshape(N_FINE),
            'wss': wss,
            'cd': cd.reshape(1),
            'cl': cl.reshape(1)}


# speedup vs baseline: 6.9842x; 1.0545x over previous
import functools

import jax
import jax.numpy as jnp
from jax import lax
from jax.experimental import pallas as pl
from jax.experimental.pallas import tpu as pltpu
from jax.experimental.pallas import tpu_sc as plsc

F32 = jnp.float32
I32 = jnp.int32

N_FINE = 10000
N_COARSE = 1000
E_FINE = 320000
E_COARSE = 32000
FD_RAW = 96
FD = 96

_SC_PARAMS = None


def _sc_compiler_params():
    return pltpu.CompilerParams(use_tc_tiling_on_sc=False)

NCORE = 2
NSUB = 16
NW = NCORE * NSUB

FK = 128
F_CHUNKS0 = 105
F_CHUNKS1 = 53
EF_PAD = NSUB * FK * (F_CHUNKS0 + F_CHUNKS1)

CK = 128
C_CHUNKS = 8
EC_PER_W = CK * C_CHUNKS
EC_PAD = EC_PER_W * NW

A_ROWS = 62500
A_ROWS_PAD = 62592
A_PER_SUB = A_ROWS_PAD // NSUB

PK = 64
P_CHUNKS = 5
NP_PER_W = PK * P_CHUNKS
NF_PAD = NP_PER_W * NW
NC_PAD = 1024

def _mm(a, b, precision=None):
    return jnp.dot(a, b, precision=precision, preferred_element_type=F32)


def _relu(v):
    return jnp.maximum(v, 0.0)


def _prep_body(ang_f, t0_f, t1_f, ang_c, t0_c, t1_c, src_c, dst_c, x,
               gf, gcos, gc, rid, lane, x16):
    a = ang_f[...]
    gf[...] = t0_f[...] * jnp.cos(a) + t1_f[...] * jnp.sin(a)
    gcos[...] = jnp.cos(a)
    ac = ang_c[...]
    gc[...] = t0_c[...] * jnp.cos(ac) + t1_c[...] * jnp.sin(ac)
    flat = dst_c[...] * N_COARSE + src_c[...]
    r = lax.shift_right_logical(flat, 4)
    rid[...] = r
    lane[...] = flat - (r * 16)
    xv = x[...]
    x16[...] = jnp.concatenate(
        [xv, jnp.ones((N_FINE, 1), F32), jnp.zeros((N_FINE, 11), F32)],
        axis=1)


_prep_call = pl.pallas_call(
    _prep_body,
    out_shape=(
        jax.ShapeDtypeStruct((E_FINE // 128, 128), F32),
        jax.ShapeDtypeStruct((E_FINE // 128, 128), F32),
        jax.ShapeDtypeStruct((E_COARSE // 128, 128), F32),
        jax.ShapeDtypeStruct((E_COARSE // 128, 128), I32),
        jax.ShapeDtypeStruct((E_COARSE // 128, 128), I32),
        jax.ShapeDtypeStruct((N_FINE, 16), F32),
    ),
)


@functools.cache
def _sc_mesh():
    return plsc.VectorSubcoreMesh(
        core_axis_name="c", subcore_axis_name="s",
        num_cores=NCORE, num_subcores=NSUB)


def _zero_fill(buf, n_rows, width):
    zer = jnp.zeros((16,), F32)

    def body(r, _):
        for q in range(width // 16):
            buf[r, pl.ds(q * 16, 16)] = zer
        return 0

    lax.fori_loop(0, n_rows, body, 0)


def _pool_body(x16_hbm, cidx_hbm, out_hbm, idx_v, rows_v, zero_v, acc_sh, sem):
    c = lax.axis_index("c")
    s = lax.axis_index("s")
    w = c * NSUB + s
    _zero_fill(zero_v, PK, 16)
    pltpu.sync_copy(zero_v, acc_sh.at[pl.ds(s * PK, PK)])
    plsc.subcore_barrier()
    pltpu.sync_copy(cidx_hbm.at[w], idx_v)

    def chunk(j, _):
        pltpu.async_copy(
            x16_hbm.at[pl.ds(w * NP_PER_W + j * PK, PK)], rows_v, sem).wait()
        pltpu.sync_copy(rows_v, acc_sh.at[idx_v.at[j]], add=True)
        return 0

    lax.fori_loop(0, P_CHUNKS, chunk, 0)
    plsc.subcore_barrier()
    pltpu.sync_copy(acc_sh.at[pl.ds(s * PK, PK)],
                    out_hbm.at[c, pl.ds(s * PK, PK)])


@functools.cache
def _pool_call():
    return functools.partial(
        pl.kernel,
        out_type=jax.ShapeDtypeStruct((NCORE, NC_PAD, 16), F32),
        mesh=_sc_mesh(),
        compiler_params=_sc_compiler_params(),
        scratch_types=[
            pltpu.VMEM((P_CHUNKS, PK), I32),
            pltpu.VMEM((PK, 16), F32),
            pltpu.VMEM((PK, 16), F32),
            pltpu.VMEM_SHARED((NC_PAD, 16), F32),
            pltpu.SemaphoreType.DMA,
        ],
    )(_pool_body)


def _adj_body(rid_hbm, lane_hbm, gate_hbm, out_hbm,
              rid_v, lane_v, gate_v, rows_v, zero_v, acc_sh, sem):
    c = lax.axis_index("c")
    s = lax.axis_index("s")
    w = c * NSUB + s
    _zero_fill(zero_v, CK, 16)
    base = s * A_PER_SUB

    def zrow(t, _):
        pltpu.sync_copy(zero_v, acc_sh.at[pl.ds(base + t * CK, CK)])
        return 0

    lax.fori_loop(0, 30, zrow, 0)
    pltpu.sync_copy(zero_v.at[pl.ds(0, 72)],
                    acc_sh.at[pl.ds(base + 30 * CK, 72)])
    plsc.subcore_barrier()

    pltpu.sync_copy(rid_hbm.at[w], rid_v)
    pltpu.sync_copy(lane_hbm.at[w], lane_v)
    pltpu.sync_copy(gate_hbm.at[w], gate_v)
    iota16 = lax.iota(I32, 16)
    zeros16 = jnp.zeros((16,), F32)

    def chunk(j, _):
        def group(t, _):
            l16 = lane_v[j, pl.ds(t * 16, 16)]
            g16 = gate_v[j, pl.ds(t * 16, 16)]
            for e in range(16):
                l_spl = jnp.full((16,), l16[e], I32)
                g_spl = jnp.full((16,), g16[e], F32)
                rows_v[t * 16 + e, :] = jnp.where(iota16 == l_spl, g_spl, zeros16)
            return 0

        lax.fori_loop(0, CK // 16, group, 0)
        pltpu.sync_copy(rows_v, acc_sh.at[rid_v.at[j]], add=True)
        return 0

    lax.fori_loop(0, C_CHUNKS, chunk, 0)
    plsc.subcore_barrier()

    def crow(t, _):
        pltpu.sync_copy(acc_sh.at[pl.ds(base + t * CK, CK)],
                        out_hbm.at[c, pl.ds(base + t * CK, CK)])
        return 0

    lax.fori_loop(0, 30, crow, 0)
    pltpu.sync_copy(acc_sh.at[pl.ds(base + 30 * CK, 72)],
                    out_hbm.at[c, pl.ds(base + 30 * CK, 72)])


@functools.cache
def _adj_call():
    return functools.partial(
        pl.kernel,
        out_type=jax.ShapeDtypeStruct((NCORE, A_ROWS_PAD, 16), F32),
        mesh=_sc_mesh(),
        compiler_params=_sc_compiler_params(),
        scratch_types=[
            pltpu.VMEM((C_CHUNKS, CK), I32),
            pltpu.VMEM((C_CHUNKS, CK), I32),
            pltpu.VMEM((C_CHUNKS, CK), F32),
            pltpu.VMEM((CK, 16), F32),
            pltpu.VMEM((CK, 16), F32),
            pltpu.VMEM_SHARED((A_ROWS_PAD, 16), F32),
            pltpu.SemaphoreType.DMA,
        ],
    )(_adj_body)


NF_ACC = 10112
_F_PER_SUB = NF_ACC // NSUB


def _fagg_body(h_hbm, src_hbm, dst_hbm, gate_hbm, out_hbm,
               src_v, dst_v, gate_v, rows_v, rows_b, acc_sh, sem, sem_b):
    c = lax.axis_index("c")
    s = lax.axis_index("s")
    w = c * NSUB + s
    _zero_fill(rows_v, 128, FD)
    base = s * _F_PER_SUB

    def zrow(t, _):
        pltpu.sync_copy(rows_v, acc_sh.at[pl.ds(base + t * 128, 128)])
        return 0

    lax.fori_loop(0, 4, zrow, 0)
    pltpu.sync_copy(rows_v.at[pl.ds(0, 120)],
                    acc_sh.at[pl.ds(base + 4 * 128, 120)])
    plsc.subcore_barrier()

    pltpu.sync_copy(src_hbm.at[w], src_v)
    pltpu.sync_copy(dst_hbm.at[w], dst_v)
    pltpu.sync_copy(gate_hbm.at[w], gate_v)

    def scale_and_scatter(j, rows, sem_g):
        pltpu.make_async_copy(h_hbm.at[src_v.at[j]], rows, sem_g).wait()

        def group(t, _):
            g16 = gate_v[j, pl.ds(t * 16, 16)]
            for e in range(16):
                g = jnp.full((16,), g16[e], F32)
                r = t * 16 + e
                for q in range(FD // 16):
                    sl = pl.ds(q * 16, 16)
                    rows[r, sl] = rows[r, sl] * g
            return 0

        lax.fori_loop(0, FK // 16, group, 0)
        pltpu.sync_copy(rows, acc_sh.at[dst_v.at[j]], add=True)

    nc = jnp.where(c == 0, F_CHUNKS0, F_CHUNKS1)
    pltpu.async_copy(h_hbm.at[src_v.at[0]], rows_v, sem)

    def pair(k, _):
        a = 2 * k
        pltpu.async_copy(h_hbm.at[src_v.at[a + 1]], rows_b, sem_b)
        scale_and_scatter(a, rows_v, sem)
        pltpu.async_copy(h_hbm.at[src_v.at[a + 2]], rows_v, sem)
        scale_and_scatter(a + 1, rows_b, sem_b)
        return 0

    lax.fori_loop(0, (nc - 1) // 2, pair, 0)
    scale_and_scatter(nc - 1, rows_v, sem)
    plsc.subcore_barrier()

    def crow(t, _):
        pltpu.sync_copy(acc_sh.at[pl.ds(base + t * 128, 128)],
                        out_hbm.at[c, pl.ds(base + t * 128, 128)])
        return 0

    lax.fori_loop(0, 4, crow, 0)
    pltpu.sync_copy(acc_sh.at[pl.ds(base + 4 * 128, 120)],
                    out_hbm.at[c, pl.ds(base + 4 * 128, 120)])


@functools.cache
def _fagg_call():
    return functools.partial(
        pl.kernel,
        out_type=jax.ShapeDtypeStruct((NCORE, NF_ACC, FD), F32),
        mesh=_sc_mesh(),
        compiler_params=_sc_compiler_params(),
        scratch_types=[
            pltpu.VMEM((F_CHUNKS0, FK), I32),
            pltpu.VMEM((F_CHUNKS0, FK), I32),
            pltpu.VMEM((F_CHUNKS0, FK), F32),
            pltpu.VMEM((FK, FD), F32),
            pltpu.VMEM((FK, FD), F32),
            pltpu.VMEM_SHARED((NF_ACC, FD), F32),
            pltpu.SemaphoreType.DMA,
            pltpu.SemaphoreType.DMA,
        ],
    )(_fagg_body)


def _coarse_body(pooled, a0, a1, W_embed, b_embed,
                 Ws0, Wn0, b0, Ws1, Wn1, b1, Ws2, Wn2, b2,
                 Ws3, Wn3, b3, Ws4, Wn4, b4, Ws5, Wn5, b5,
                 W_csb, b_csb, W_cd1, b_cd1, W_cd2, b_cd2,
                 W_cl1, b_cl1, W_cl2, b_cl2, W_up192,
                 w1_out, cd_out, cl_out):
    ps = pooled[0] + pooled[1]
    sums4 = ps[0:N_COARSE, 0:4]
    cnt_raw = ps[0:N_COARSE, 4:5]
    cnt = jnp.maximum(cnt_raw, 1.0)
    h = _mm(sums4, W_embed[...]) / cnt + jnp.minimum(cnt_raw, 1.0) * b_embed[...]
    A = a0[...] + a1[...]
    blocks = ((Ws0, Wn0, b0), (Ws1, Wn1, b1), (Ws2, Wn2, b2),
              (Ws3, Wn3, b3), (Ws4, Wn4, b4), (Ws5, Wn5, b5))
    for Ws, Wn, b in blocks:
        din, dout = Ws.shape
        if dout <= din:
            agg = _mm(A, _mm(h, Wn[...]))
        else:
            agg = _mm(_mm(A, h), Wn[...])
        h = _relu(_mm(h, Ws[...]) + agg + b[...])
    h_cs = _mm(h, W_csb[...]) + b_csb[...]
    pv = jnp.mean(h_cs, axis=0, keepdims=True)
    cd_out[...] = _mm(_relu(_mm(pv, W_cd1[...]) + b_cd1[...]), W_cd2[...]) + b_cd2[...]
    cl_out[...] = _mm(_relu(_mm(pv, W_cl1[...]) + b_cl1[...]), W_cl2[...]) + b_cl2[...]
    w1_out[...] = _mm(h, W_up192[...])


_coarse_call = pl.pallas_call(
    _coarse_body,
    out_shape=(
        jax.ShapeDtypeStruct((N_COARSE, FD_RAW), F32),
        jax.ShapeDtypeStruct((1, 1), F32),
        jax.ShapeDtypeStruct((1, 1), F32),
    ),
)


_IROWS = 1000


def _interp_body(im, x16, w1, w2, b_up, out):
    acc = _mm(im[...], w1[...], precision=lax.Precision.DEFAULT)
    out[...] = _relu(acc + _mm(x16[...], w2[...]) + b_up[...])


_interp_call = pl.pallas_call(
    _interp_body,
    grid=(N_FINE // _IROWS,),
    in_specs=[
        pl.BlockSpec((_IROWS, N_COARSE), lambda i: (i, 0)),
        pl.BlockSpec((_IROWS, 16), lambda i: (i, 0)),
        pl.BlockSpec((N_COARSE, FD), lambda i: (0, 0)),
        pl.BlockSpec((16, FD), lambda i: (0, 0)),
        pl.BlockSpec((1, FD), lambda i: (0, 0)),
    ],
    out_specs=pl.BlockSpec((_IROWS, FD), lambda i: (i, 0)),
    out_shape=jax.ShapeDtypeStruct((N_FINE, FD), F32),
)


def _fupd_body(h, p0, p1, Ws, Wn, b, out):
    out[...] = _relu(_mm(h[...], Ws[...]) + _mm(p0[...] + p1[...], Wn[...]) + b[...])


_FR = 2000

_fupd_call = pl.pallas_call(
    _fupd_body,
    grid=(N_FINE // _FR,),
    in_specs=[
        pl.BlockSpec((_FR, FD), lambda i: (i, 0)),
        pl.BlockSpec((_FR, FD), lambda i: (i, 0)),
        pl.BlockSpec((_FR, FD), lambda i: (i, 0)),
        pl.BlockSpec((FD, FD), lambda i: (0, 0)),
        pl.BlockSpec((FD, FD), lambda i: (0, 0)),
        pl.BlockSpec((1, FD), lambda i: (0, 0)),
    ],
    out_specs=pl.BlockSpec((_FR, FD), lambda i: (i, 0)),
    out_shape=jax.ShapeDtypeStruct((N_FINE, FD), F32),
)


def _heads_body(h, p0, p1, W_psb, b_psb, W_cp1, b_cp1, W_cp2, b_cp2,
                W_wss, e1, e2, cp_out, wss_out):
    h_cp = _mm(h[...], W_psb[...]) + b_psb[...]
    cp_out[...] = _mm(_relu(_mm(h_cp, W_cp1[...]) + b_cp1[...]), W_cp2[...]) + b_cp2[...]
    coeffs = _mm(p0[...] + p1[...], W_wss[...])
    wss_out[...] = coeffs[:, 0:1] * e1[...] + coeffs[:, 1:2] * e2[...]


_heads_call = pl.pallas_call(
    _heads_body,
    grid=(N_FINE // _FR,),
    in_specs=[
        pl.BlockSpec((_FR, FD), lambda i: (i, 0)),
        pl.BlockSpec((_FR, FD), lambda i: (i, 0)),
        pl.BlockSpec((_FR, FD), lambda i: (i, 0)),
        pl.BlockSpec((FD, 64), lambda i: (0, 0)),
        pl.BlockSpec((1, 64), lambda i: (0, 0)),
        pl.BlockSpec((64, 128), lambda i: (0, 0)),
        pl.BlockSpec((1, 128), lambda i: (0, 0)),
        pl.BlockSpec((128, 1), lambda i: (0, 0)),
        pl.BlockSpec((1, 1), lambda i: (0, 0)),
        pl.BlockSpec((FD, 2), lambda i: (0, 0)),
        pl.BlockSpec((_FR, 3), lambda i: (i, 0)),
        pl.BlockSpec((_FR, 3), lambda i: (i, 0)),
    ],
    out_specs=(
        pl.BlockSpec((_FR, 1), lambda i: (i, 0)),
        pl.BlockSpec((_FR, 3), lambda i: (i, 0)),
    ),
    out_shape=(
        jax.ShapeDtypeStruct((N_FINE, 1), F32),
        jax.ShapeDtypeStruct((N_FINE, 3), F32),
    ),
)


def _pad_worker(arr, per_w, chunks, k, dtype):
    n = arr.shape[0]
    arr = jnp.pad(arr, (0, per_w * NW - n))
    return arr.reshape(NW, chunks, k).astype(dtype)


def _pack_fine(arr, dtype):
    n = arr.shape[0]
    arr = jnp.pad(arr, (0, EF_PAD - n)).astype(dtype)
    n0 = NSUB * F_CHUNKS0 * FK
    p0 = arr[:n0].reshape(NSUB, F_CHUNKS0, FK)
    p1 = arr[n0:].reshape(NSUB, F_CHUNKS1, FK)
    p1 = jnp.pad(p1, ((0, 0), (0, F_CHUNKS0 - F_CHUNKS1), (0, 0)))
    return jnp.concatenate([p0, p1], axis=0)


def kernel(x, fine_edge_index, fine_angles, fine_transporters, coarse_idx,
           coarse_edge_index, coarse_angles, coarse_transporters, interp_matrix,
           e1, e2, W_embed, b_embed,
           Wc_self_0, Wc_nbr_0, b_c_0, Wc_self_1, Wc_nbr_1, b_c_1,
           Wc_self_2, Wc_nbr_2, b_c_2, Wc_self_3, Wc_nbr_3, b_c_3,
           Wc_self_4, Wc_nbr_4, b_c_4, Wc_self_5, Wc_nbr_5, b_c_5,
           W_csb, b_csb, W_cd1, b_cd1, W_cd2, b_cd2,
           W_cl1, b_cl1, W_cl2, b_cl2, W_up, b_up,
           Wr_self_0, Wr_nbr_0, b_r_0, Wr_self_1, Wr_nbr_1, b_r_1,
           W_psb, b_psb, W_cp1, b_cp1, W_cp2, b_cp2, W_wss):
    row = lambda v: v.reshape(1, -1)

    ang_f = fine_angles.reshape(E_FINE // 128, 128)
    t0_f = fine_transporters[:, 0].reshape(E_FINE // 128, 128)
    t1_f = fine_transporters[:, 1].reshape(E_FINE // 128, 128)
    ang_c = coarse_angles.reshape(E_COARSE // 128, 128)
    t0_c = coarse_transporters[:, 0].reshape(E_COARSE // 128, 128)
    t1_c = coarse_transporters[:, 1].reshape(E_COARSE // 128, 128)
    src_c = coarse_edge_index[0].reshape(E_COARSE // 128, 128)
    dst_c = coarse_edge_index[1].reshape(E_COARSE // 128, 128)
    gf, gcos, gc, rid, lane, x16 = _prep_call(
        ang_f, t0_f, t1_f, ang_c, t0_c, t1_c, src_c, dst_c, x)

    x16_pad = jnp.pad(x16, ((0, NF_PAD - N_FINE), (0, 0)))
    cidx = _pad_worker(coarse_idx, NP_PER_W, P_CHUNKS, PK, I32)
    pooled = _pool_call()(x16_pad, cidx)

    rid_w = _pad_worker(rid.reshape(-1), EC_PER_W, C_CHUNKS, CK, I32)
    lane_w = _pad_worker(lane.reshape(-1), EC_PER_W, C_CHUNKS, CK, I32)
    gc_w = _pad_worker(gc.reshape(-1), EC_PER_W, C_CHUNKS, CK, F32)
    a_parts = _adj_call()(rid_w, lane_w, gc_w)
    a_flat = a_parts.reshape(NCORE, A_ROWS_PAD * 16)[:, :N_COARSE * N_COARSE]
    a0 = a_flat[0].reshape(N_COARSE, N_COARSE)
    a1 = a_flat[1].reshape(N_COARSE, N_COARSE)

    w1, cd, cl = _coarse_call(
        pooled, a0, a1, W_embed, row(b_embed),
        Wc_self_0, Wc_nbr_0, row(b_c_0), Wc_self_1, Wc_nbr_1, row(b_c_1),
        Wc_self_2, Wc_nbr_2, row(b_c_2), Wc_self_3, Wc_nbr_3, row(b_c_3),
        Wc_self_4, Wc_nbr_4, row(b_c_4), Wc_self_5, Wc_nbr_5, row(b_c_5),
        W_csb, row(b_csb), W_cd1, row(b_cd1), W_cd2, row(b_cd2),
        W_cl1, row(b_cl1), W_cl2, row(b_cl2), W_up[0:192, :])

    pad_c = FD - FD_RAW
    w1p = jnp.pad(w1, ((0, 0), (0, pad_c)))
    w2 = jnp.pad(W_up[192:196, :], ((0, 12), (0, pad_c)))
    h_f = _interp_call(interp_matrix, x16, w1p, w2, row(jnp.pad(b_up, (0, pad_c))))

    src_w = _pack_fine(fine_edge_index[0], I32)
    dst_w = _pack_fine(fine_edge_index[1], I32)
    gf_w = _pack_fine(gf.reshape(-1), F32)
    for Ws, Wn, b in ((Wr_self_0, Wr_nbr_0, b_r_0),
                      (Wr_self_1, Wr_nbr_1, b_r_1)):
        parts = _fagg_call()(h_f, src_w, dst_w, gf_w)
        h_f = _fupd_call(h_f, parts[0, :N_FINE], parts[1, :N_FINE],
                         jnp.pad(Ws, ((0, pad_c), (0, pad_c))),
                         jnp.pad(Wn, ((0, 0), (0, pad_c))),
                         row(jnp.pad(b, (0, pad_c))))

    gcos_w = _pack_fine(gcos.reshape(-1), F32)
    parts = _fagg_call()(h_f, src_w, dst_w, gcos_w)

    cp, wss = _heads_call(
        h_f, parts[0, :N_FINE], parts[1, :N_FINE],
        jnp.pad(W_psb, ((0, pad_c), (0, 0))), row(b_psb), W_cp1, row(b_cp1),
        W_cp2, row(b_cp2), W_wss, e1, e2)

    return {'cp': cp.reshape(N_FINE),
            'wss': wss,
            'cd': cd.reshape(1),
            'cl': cl.reshape(1)}
